# Initial kernel scaffold; baseline (speedup 1.0000x reference)
import jax
import jax.numpy as jnp
from jax.experimental import pallas as pl


def kernel(idx, tok_table, pos_table):
    raise NotImplementedError("write your pallas kernel here")


# speedup vs baseline: 1.1679x; 1.1679x over previous
import functools

import jax
import jax.numpy as jnp
from jax import lax
from jax.experimental import pallas as pl
from jax.experimental.pallas import tpu as pltpu
from jax.experimental.pallas import tpu_sc as plsc

VOCAB = 100000
D = 768
B = 4
T = 2048

_info = plsc.get_sparse_core_info()
NC, NS, L = _info.num_cores, _info.num_subcores, _info.num_lanes
NW = NC * NS
TS = T // NW
DV = D // L


def _emb_kernel(idx_hbm, tok_hbm, pos_hbm, out_hbm, idx_v, rows_v, pos_v, sem):
    wid = lax.axis_index("s") * NC + lax.axis_index("c")
    t0 = wid * TS

    pltpu.sync_copy(pos_hbm.at[pl.ds(t0, TS)], pos_v)

    for b in range(B):
        base = b * T + t0
        pltpu.sync_copy(idx_hbm.at[pl.ds(base, TS)], idx_v)
        pltpu.async_copy(tok_hbm.at[idx_v], rows_v, sem).wait()

        def add_row(r, _):
            for d in range(DV):
                sl = pl.ds(d * L, L)
                rows_v[r, sl] = rows_v[r, sl] + pos_v[r, sl]
            return _

        lax.fori_loop(0, TS, add_row, 0)
        pltpu.sync_copy(rows_v, out_hbm.at[pl.ds(base, TS)])


@jax.jit
def _emb(idx_flat, tok_table, pos_table):
    mesh = plsc.VectorSubcoreMesh(core_axis_name="c", subcore_axis_name="s")
    run = functools.partial(
        pl.kernel,
        mesh=mesh,
        out_type=jax.ShapeDtypeStruct((B * T, D), jnp.float32),
        scratch_types=[
            pltpu.VMEM((TS,), jnp.int32),
            pltpu.VMEM((TS, D), jnp.float32),
            pltpu.VMEM((TS, D), jnp.float32),
            pltpu.SemaphoreType.DMA,
        ],
    )(_emb_kernel)
    return run(idx_flat, tok_table, pos_table)


def kernel(idx, tok_table, pos_table):
    idx_flat = idx.reshape(-1).astype(jnp.int32)
    out = _emb(idx_flat, tok_table, pos_table)
    return out.reshape(idx.shape[0], idx.shape[1], D)


# speedup vs baseline: 1.3373x; 1.1451x over previous
import functools

import jax
import jax.numpy as jnp
from jax import lax
from jax.experimental import pallas as pl
from jax.experimental.pallas import tpu as pltpu
from jax.experimental.pallas import tpu_sc as plsc

VOCAB = 100000
D = 768
B = 4
T = 2048

_info = plsc.get_sparse_core_info()
NC, NS, L = _info.num_cores, _info.num_subcores, _info.num_lanes
NW = NC * NS
TS = T // NW
DV = D // L
TC = 8
NCH = TS // TC
NBUF = 4


def _emb_kernel(idx_hbm, tok_hbm, pos_hbm, out_hbm, idx_v,
                rows0, rows1, rows2, rows3, pos0, pos1, pos2, pos3,
                g0, g1, g2, g3, o0, o1, o2, o3):
    wid = lax.axis_index("s") * NC + lax.axis_index("c")
    t0 = wid * TS

    rows = [rows0, rows1, rows2, rows3]
    pos = [pos0, pos1, pos2, pos3]
    gsem = [g0, g1, g2, g3]
    osem = [o0, o1, o2, o3]

    for b in range(B):
        pltpu.sync_copy(idx_hbm.at[pl.ds(b * T + t0, TS)], idx_v.at[b])

    gather_descs = [None] * NCH
    out_descs = [None] * NCH

    def start_gather(c):
        s = c % NBUF
        descs = [pltpu.async_copy(
            pos_hbm.at[pl.ds(t0 + c * TC, TC)], pos[s], gsem[s])]
        for b in range(B):
            descs.append(pltpu.async_copy(
                tok_hbm.at[idx_v.at[b, pl.ds(c * TC, TC)]],
                rows[s].at[b], gsem[s]))
        gather_descs[c] = descs

    def start_out(c):
        s = c % NBUF
        out_descs[c] = [pltpu.async_copy(
            rows[s].at[b],
            out_hbm.at[pl.ds(b * T + t0 + c * TC, TC)],
            osem[s]) for b in range(B)]

    for c in range(NBUF):
        start_gather(c)

    for c in range(NCH):
        s = c % NBUF
        for dsc in gather_descs[c]:
            dsc.wait()

        rbuf, pbuf = rows[s], pos[s]

        def add_row(r, _):
            for d in range(DV):
                sl = pl.ds(d * L, L)
                p = pbuf[r, sl]
                for b in range(B):
                    rbuf[b, r, sl] = rbuf[b, r, sl] + p
            return _

        lax.fori_loop(0, TC, add_row, 0)

        if c >= 1:
            for dsc in out_descs[c - 1]:
                dsc.wait()
        if c + NBUF < NCH:
            start_gather(c + NBUF)
        start_out(c)

    for dsc in out_descs[NCH - 1]:
        dsc.wait()


@jax.jit
def _emb(idx_flat, tok_table, pos_table):
    mesh = plsc.VectorSubcoreMesh(core_axis_name="c", subcore_axis_name="s")
    run = functools.partial(
        pl.kernel,
        mesh=mesh,
        out_type=jax.ShapeDtypeStruct((B * T, D), jnp.float32),
        scratch_types=(
            [pltpu.VMEM((B, TS), jnp.int32)]
            + [pltpu.VMEM((B, TC, D), jnp.float32)] * NBUF
            + [pltpu.VMEM((TC, D), jnp.float32)] * NBUF
            + [pltpu.SemaphoreType.DMA] * (2 * NBUF)
        ),
    )(_emb_kernel)
    return run(idx_flat, tok_table, pos_table)


def kernel(idx, tok_table, pos_table):
    idx_flat = idx.reshape(-1).astype(jnp.int32)
    out = _emb(idx_flat, tok_table, pos_table)
    return out.reshape(idx.shape[0], idx.shape[1], D)


# speedup vs baseline: 1.3796x; 1.0316x over previous
import functools

import jax
import jax.numpy as jnp
from jax import lax
from jax.experimental import pallas as pl
from jax.experimental.pallas import tpu as pltpu
from jax.experimental.pallas import tpu_sc as plsc

VOCAB = 100000
D = 768
B = 4
T = 2048

_info = plsc.get_sparse_core_info()
NC, NS, L = _info.num_cores, _info.num_subcores, _info.num_lanes
NW = NC * NS
TS = T // NW
DV = D // L
TC = 8
NCH = TS // TC
NBUF = 3


def _emb_kernel(idx_hbm, tok_hbm, pos_hbm, out_hbm, idx_v, pos_v,
                rows0, rows1, rows2, g0, g1, g2, o0, o1, o2, psem):
    wid = lax.axis_index("s") * NC + lax.axis_index("c")
    t0 = wid * TS

    rows = [rows0, rows1, rows2]
    gsem = [g0, g1, g2]
    osem = [o0, o1, o2]

    pos_dsc = pltpu.async_copy(pos_hbm.at[pl.ds(t0, TS)], pos_v, psem)
    for b in range(B):
        pltpu.sync_copy(idx_hbm.at[b, pl.ds(t0, TS)], idx_v.at[b])

    gather_descs = [None] * NCH
    out_descs = [None] * NCH

    def start_gather(c):
        s = c % NBUF
        gather_descs[c] = [pltpu.async_copy(
            tok_hbm.at[idx_v.at[b, pl.ds(c * TC, TC)]],
            rows[s].at[b], gsem[s]) for b in range(B)]

    def start_out(c):
        s = c % NBUF
        out_descs[c] = [pltpu.async_copy(
            rows[s].at[b],
            out_hbm.at[b, pl.ds(t0 + c * TC, TC)],
            osem[s]) for b in range(B)]

    for c in range(NBUF):
        start_gather(c)
    pos_dsc.wait()

    for c in range(NCH):
        s = c % NBUF
        for dsc in gather_descs[c]:
            dsc.wait()

        rbuf = rows[s]

        def add_row(r, _):
            for d in range(DV):
                sl = pl.ds(d * L, L)
                p = pos_v[c * TC + r, sl]
                for b in range(B):
                    rbuf[b, r, sl] = rbuf[b, r, sl] + p
            return _

        lax.fori_loop(0, TC, add_row, 0)

        if c >= 1:
            for dsc in out_descs[c - 1]:
                dsc.wait()
        if c + NBUF < NCH:
            start_gather(c + NBUF)
        start_out(c)

    for dsc in out_descs[NCH - 1]:
        dsc.wait()


@jax.jit
def _emb(idx, tok_table, pos_table):
    mesh = plsc.VectorSubcoreMesh(core_axis_name="c", subcore_axis_name="s")
    run = functools.partial(
        pl.kernel,
        mesh=mesh,
        out_type=jax.ShapeDtypeStruct((B, T, D), jnp.float32),
        scratch_types=(
            [pltpu.VMEM((B, TS), jnp.int32),
             pltpu.VMEM((TS, D), jnp.float32)]
            + [pltpu.VMEM((B, TC, D), jnp.float32)] * NBUF
            + [pltpu.SemaphoreType.DMA] * (2 * NBUF + 1)
        ),
    )(_emb_kernel)
    return run(idx, tok_table, pos_table)


def kernel(idx, tok_table, pos_table):
    return _emb(idx.astype(jnp.int32), tok_table, pos_table)
